# baseline (speedup 1.0000x reference)
import functools

import jax
import jax.numpy as jnp
from jax import lax
from jax.experimental import pallas as pl
from jax.experimental.pallas import tpu as pltpu
from jax.experimental.pallas import tpu_sc as plsc

B = 16384
D = 64
NC = 2
NS = 16
L = 16
NW = NC * NS
BPW = B // NW
EU = 2 * B // NW
RU = B // NW
EB = 128
NEB = (1000000 + EB - 1) // EB


def _unpack(v):
    return v & 0xFFFF, v >> 16


def _user_section(idx, pfl, nxt, nusers, tre_hbm, tim_hbm, out_re, out_im,
                  bre, bim, rbre, rbim, bsem, osem):
    def first_block(uv0):
        return uv0 >> 7

    iv0 = plsc.load_gather(idx, [jnp.zeros((L,), jnp.int32)])
    b0 = first_block(iv0[0])
    off0 = pl.multiple_of(b0 * EB, EB)
    cp0 = pltpu.async_copy(tre_hbm.at[:, pl.ds(off0, EB)], bre.at[0], bsem)
    cp1 = pltpu.async_copy(tim_hbm.at[:, pl.ds(off0, EB)], bim.at[0], bsem)

    def user_step(u, cnt):
        uu = jnp.full((L,), u, jnp.int32)
        i = plsc.load_gather(idx, [uu])[0]
        pf = plsc.load_gather(pfl, [uu])[0]
        nx = plsc.load_gather(nxt, [uu])[0]
        pos, flag = _unpack(pf)
        boundary = jnp.logical_or(u == 0, flag == 1)

        def on_boundary():
            pltpu.make_async_copy(
                tre_hbm.at[:, pl.ds(0, EB)], bre.at[0], bsem).wait()
            pltpu.make_async_copy(
                tim_hbm.at[:, pl.ds(0, EB)], bim.at[0], bsem).wait()
            nslot = (cnt + 1) & 1
            noff = pl.multiple_of(nx * EB, EB)
            for s in range(2):
                @pl.when(nslot == s)
                def _(s=s):
                    pltpu.async_copy(tre_hbm.at[:, pl.ds(noff, EB)],
                                     bre.at[s], bsem)
                    pltpu.async_copy(tim_hbm.at[:, pl.ds(noff, EB)],
                                     bim.at[s], bsem)

        pl.when(boundary)(on_boundary)
        cnt = jnp.where(boundary, cnt + 1, cnt)
        slot = (cnt - 1) & 1

        imod = i & (EB - 1)
        sl16 = jnp.full((L,), slot, jnp.int32)
        im16 = jnp.full((L,), imod, jnp.int32)
        ubase = (u & 63) * (2 * D)
        for dc in range(D // L):
            dvec = lax.iota(jnp.int32, L) + dc * L
            cre = plsc.load_gather(bre, [sl16, dvec, im16])
            cim = plsc.load_gather(bim, [sl16, dvec, im16])
            rbre[pl.ds(ubase + dc * L, L)] = cre
            rbim[pl.ds(ubase + dc * L, L)] = cim
        pltpu.async_copy(rbre.at[pl.ds(ubase, 2 * D)], out_re.at[pos], osem)
        pltpu.async_copy(rbim.at[pl.ds(ubase, 2 * D)], out_im.at[pos], osem)
        return cnt

    lax.fori_loop(0, nusers, user_step, jnp.int32(0))
    pltpu.make_async_copy(tre_hbm.at[:, pl.ds(0, EB)], bre.at[0], bsem).wait()
    pltpu.make_async_copy(tim_hbm.at[:, pl.ds(0, EB)], bim.at[0], bsem).wait()
    for _ in range(nusers // 32):
        pltpu.make_async_copy(
            tre_hbm.at[:, pl.ds(0, EB)], bre.at[0], osem).wait()


def _phase1_body(si, pf, nx, sr, rpf, rnx, ere_t, eim_t, rre_t, rim_t,
                 rows_re, rows_im, rel_re, rel_im,
                 iv, pv, nv, riv, rpv, rnv, bre, bim, rbre, rbim,
                 bsem, osem):
    wid = lax.axis_index("s") * NC + lax.axis_index("c")
    pltpu.sync_copy(si.at[pl.ds(wid * EU, EU)], iv)
    pltpu.sync_copy(pf.at[pl.ds(wid * EU, EU)], pv)
    pltpu.sync_copy(nx.at[pl.ds(wid * EU, EU)], nv)
    pltpu.sync_copy(sr.at[pl.ds(wid * RU, RU)], riv)
    pltpu.sync_copy(rpf.at[pl.ds(wid * RU, RU)], rpv)
    pltpu.sync_copy(rnx.at[pl.ds(wid * RU, RU)], rnv)
    _user_section(iv, pv, nv, EU, ere_t, eim_t, rows_re, rows_im,
                  bre, bim, rbre, rbim, bsem, osem)
    _user_section(riv, rpv, rnv, RU, rre_t, rim_t, rel_re, rel_im,
                  bre, bim, rbre, rbim, bsem, osem)


def _phase2_body(rows_re, rows_im, rel_re, rel_im, out_hbm,
                 hre, him, tre, tim, rre, rim, scores, sem):
    wid = lax.axis_index("s") * NC + lax.axis_index("c")
    CH2 = 64
    for c in range(BPW // CH2):
        base = wid * BPW + c * CH2
        cps = [
            pltpu.async_copy(rows_re.at[pl.ds(base, CH2)], hre, sem),
            pltpu.async_copy(rows_im.at[pl.ds(base, CH2)], him, sem),
            pltpu.async_copy(rows_re.at[pl.ds(B + base, CH2)], tre, sem),
            pltpu.async_copy(rows_im.at[pl.ds(B + base, CH2)], tim, sem),
            pltpu.async_copy(rel_re.at[pl.ds(base, CH2)], rre, sem),
            pltpu.async_copy(rel_im.at[pl.ds(base, CH2)], rim, sem),
        ]
        for cp in cps:
            cp.wait()
        for g in range(CH2 // L):
            rows = lax.iota(jnp.int32, L) + (g * L)

            def dim_step(d4, acc, rows=rows):
                for u in range(4):
                    cols = d4 * 4 + jnp.full((L,), u, jnp.int32)
                    a = plsc.load_gather(hre, [rows, cols])
                    bb = plsc.load_gather(him, [rows, cols])
                    cr = plsc.load_gather(rre, [rows, cols])
                    ci = plsc.load_gather(rim, [rows, cols])
                    e = plsc.load_gather(tre, [rows, cols])
                    f = plsc.load_gather(tim, [rows, cols])
                    acc = acc + e * (a * cr - bb * ci) + f * (bb * cr + a * ci)
                return acc

            acc = lax.fori_loop(0, D // 4, dim_step,
                                jnp.zeros((L,), jnp.float32))
            scores[pl.ds(c * CH2 + g * L, L)] = acc
    pltpu.sync_copy(scores, out_hbm.at[pl.ds(wid * BPW, BPW)])


def _next_distinct(blocks, flags):
    n = blocks.shape[0]
    idxs = jnp.arange(n, dtype=jnp.int32)
    binf = jnp.where(flags == 1, idxs, jnp.int32(2**30))
    suffix_min = lax.cummin(binf, axis=0, reverse=True)
    nxt_idx = jnp.concatenate(
        [suffix_min[1:], jnp.full((1,), 2**30, jnp.int32)])
    return blocks[jnp.clip(nxt_idx, 0, n - 1)]


@functools.partial(jax.jit)
def kernel(h, r, t, entity_re, entity_im, relation_re, relation_im):
    h32 = h.astype(jnp.int32)
    t32 = t.astype(jnp.int32)
    r32 = r.astype(jnp.int32)
    keys = jnp.concatenate([h32, t32])
    order = jnp.argsort(keys).astype(jnp.int32)
    si = keys[order]
    bu = si >> 7
    flag = jnp.concatenate(
        [jnp.ones((1,), jnp.int32), (bu[1:] != bu[:-1]).astype(jnp.int32)])
    pf = order + flag * 65536
    nx = jnp.clip(_next_distinct(bu, flag), 0, NEB - 1)

    rorder = jnp.argsort(r32).astype(jnp.int32)
    sr = r32[rorder]
    rbu = sr >> 7
    rflag = jnp.concatenate(
        [jnp.ones((1,), jnp.int32), (rbu[1:] != rbu[:-1]).astype(jnp.int32)])
    rpf = rorder + rflag * 65536
    rnx = jnp.clip(_next_distinct(rbu, rflag), 0, 7)

    ere_t = entity_re.T
    eim_t = entity_im.T
    rre_t = relation_re.T
    rim_t = relation_im.T

    mesh = plsc.VectorSubcoreMesh(
        core_axis_name="c", subcore_axis_name="s", num_cores=NC,
        num_subcores=NS)
    p1 = pl.kernel(
        _phase1_body,
        out_type=(
            jax.ShapeDtypeStruct((2 * B, 2 * D), jnp.float32),
            jax.ShapeDtypeStruct((2 * B, 2 * D), jnp.float32),
            jax.ShapeDtypeStruct((B, 2 * D), jnp.float32),
            jax.ShapeDtypeStruct((B, 2 * D), jnp.float32),
        ),
        mesh=mesh,
        scratch_types=[
            pltpu.VMEM((EU,), jnp.int32),
            pltpu.VMEM((EU,), jnp.int32),
            pltpu.VMEM((EU,), jnp.int32),
            pltpu.VMEM((RU,), jnp.int32),
            pltpu.VMEM((RU,), jnp.int32),
            pltpu.VMEM((RU,), jnp.int32),
            pltpu.VMEM((2, D, EB), jnp.float32),
            pltpu.VMEM((2, D, EB), jnp.float32),
            pltpu.VMEM((64 * 2 * D,), jnp.float32),
            pltpu.VMEM((64 * 2 * D,), jnp.float32),
            pltpu.SemaphoreType.DMA,
            pltpu.SemaphoreType.DMA,
        ],
        compiler_params=pltpu.CompilerParams(
            needs_layout_passes=False, disable_bounds_checks=True),
    )
    rows_re, rows_im, rel_re, rel_im = p1(
        si, pf, nx, sr, rpf, rnx, ere_t, eim_t, rre_t, rim_t)

    p2 = pl.kernel(
        _phase2_body,
        out_type=jax.ShapeDtypeStruct((B,), jnp.float32),
        mesh=mesh,
        scratch_types=[
            pltpu.VMEM((64, 2 * D), jnp.float32),
            pltpu.VMEM((64, 2 * D), jnp.float32),
            pltpu.VMEM((64, 2 * D), jnp.float32),
            pltpu.VMEM((64, 2 * D), jnp.float32),
            pltpu.VMEM((64, 2 * D), jnp.float32),
            pltpu.VMEM((64, 2 * D), jnp.float32),
            pltpu.VMEM((BPW,), jnp.float32),
            pltpu.SemaphoreType.DMA,
        ],
        compiler_params=pltpu.CompilerParams(needs_layout_passes=False),
    )
    return p2(rows_re, rows_im, rel_re, rel_im)


# speedup vs baseline: 1.0155x; 1.0155x over previous
import functools

import jax
import jax.numpy as jnp
from jax import lax
from jax.experimental import pallas as pl
from jax.experimental.pallas import tpu as pltpu
from jax.experimental.pallas import tpu_sc as plsc

B = 16384
D = 64
NC = 2
NS = 16
L = 16
NW = NC * NS
BPW = B // NW
EU = 2 * B // NW
RU = B // NW
EB = 128
NEB = (1000000 + EB - 1) // EB


def _unpack(v):
    return v & 0xFFFF, v >> 16


def _user_section(idx, pfl, nxt, nusers, tre_hbm, tim_hbm, out_re, out_im,
                  bre, bim, rbre, rbim, bsem, osem):
    def first_block(uv0):
        return uv0 >> 7

    iv0 = idx[pl.ds(0, L)]
    b0 = first_block(iv0[0])
    off0 = pl.multiple_of(b0 * EB, EB)
    cp0 = pltpu.async_copy(tre_hbm.at[:, pl.ds(off0, EB)], bre.at[0], bsem)
    cp1 = pltpu.async_copy(tim_hbm.at[:, pl.ds(off0, EB)], bim.at[0], bsem)

    def group_step(g, cnt):
        base = g * L
        iv16 = idx[pl.ds(base, L)]
        pf16 = pfl[pl.ds(base, L)]
        nv16 = nxt[pl.ds(base, L)]
        pos16, flag16 = _unpack(pf16)
        imod16 = iv16 & (EB - 1)
        gb = (g & 3) * (L * 2 * D)
        for k in range(L):
            flag = flag16[k]
            boundary = jnp.logical_or((base == 0) & (k == 0), flag == 1)
            nx = nv16[k]

            def on_boundary(nx=nx, cnt=cnt):
                pltpu.make_async_copy(
                    tre_hbm.at[:, pl.ds(0, EB)], bre.at[0], bsem).wait()
                pltpu.make_async_copy(
                    tim_hbm.at[:, pl.ds(0, EB)], bim.at[0], bsem).wait()
                nslot = (cnt + 1) & 1
                noff = pl.multiple_of(nx * EB, EB)
                for s in range(2):
                    @pl.when(nslot == s)
                    def _(s=s):
                        pltpu.async_copy(tre_hbm.at[:, pl.ds(noff, EB)],
                                         bre.at[s], bsem)
                        pltpu.async_copy(tim_hbm.at[:, pl.ds(noff, EB)],
                                         bim.at[s], bsem)

            pl.when(boundary)(on_boundary)
            cnt = jnp.where(boundary, cnt + 1, cnt)
            slot = (cnt - 1) & 1

            sl16 = jnp.full((L,), slot, jnp.int32)
            im16 = jnp.full((L,), imod16[k], jnp.int32)
            ubase = gb + k * (2 * D)
            for dc in range(D // L):
                dvec = lax.iota(jnp.int32, L) + dc * L
                cre = plsc.load_gather(bre, [sl16, dvec, im16])
                cim = plsc.load_gather(bim, [sl16, dvec, im16])
                rbre[pl.ds(ubase + dc * L, L)] = cre
                rbim[pl.ds(ubase + dc * L, L)] = cim
            pltpu.async_copy(rbre.at[pl.ds(ubase, 2 * D)],
                             out_re.at[pos16[k]], osem)
            pltpu.async_copy(rbim.at[pl.ds(ubase, 2 * D)],
                             out_im.at[pos16[k]], osem)
        return cnt

    lax.fori_loop(0, nusers // L, group_step, jnp.int32(0))
    pltpu.make_async_copy(tre_hbm.at[:, pl.ds(0, EB)], bre.at[0], bsem).wait()
    pltpu.make_async_copy(tim_hbm.at[:, pl.ds(0, EB)], bim.at[0], bsem).wait()
    for _ in range(nusers // 32):
        pltpu.make_async_copy(
            tre_hbm.at[:, pl.ds(0, EB)], bre.at[0], osem).wait()


def _phase1_body(si, pf, nx, sr, rpf, rnx, ere_t, eim_t, rre_t, rim_t,
                 rows_re, rows_im, rel_re, rel_im,
                 iv, pv, nv, riv, rpv, rnv, bre, bim, rbre, rbim,
                 bsem, osem):
    wid = lax.axis_index("s") * NC + lax.axis_index("c")
    pltpu.sync_copy(si.at[pl.ds(wid * EU, EU)], iv)
    pltpu.sync_copy(pf.at[pl.ds(wid * EU, EU)], pv)
    pltpu.sync_copy(nx.at[pl.ds(wid * EU, EU)], nv)
    pltpu.sync_copy(sr.at[pl.ds(wid * RU, RU)], riv)
    pltpu.sync_copy(rpf.at[pl.ds(wid * RU, RU)], rpv)
    pltpu.sync_copy(rnx.at[pl.ds(wid * RU, RU)], rnv)
    _user_section(iv, pv, nv, EU, ere_t, eim_t, rows_re, rows_im,
                  bre, bim, rbre, rbim, bsem, osem)
    _user_section(riv, rpv, rnv, RU, rre_t, rim_t, rel_re, rel_im,
                  bre, bim, rbre, rbim, bsem, osem)


def _phase2_body(rows_re, rows_im, rel_re, rel_im, out_hbm,
                 hre, him, tre, tim, rre, rim, scores, sem):
    wid = lax.axis_index("s") * NC + lax.axis_index("c")
    CH2 = 64
    for c in range(BPW // CH2):
        base = wid * BPW + c * CH2
        cps = [
            pltpu.async_copy(rows_re.at[pl.ds(base, CH2)], hre, sem),
            pltpu.async_copy(rows_im.at[pl.ds(base, CH2)], him, sem),
            pltpu.async_copy(rows_re.at[pl.ds(B + base, CH2)], tre, sem),
            pltpu.async_copy(rows_im.at[pl.ds(B + base, CH2)], tim, sem),
            pltpu.async_copy(rel_re.at[pl.ds(base, CH2)], rre, sem),
            pltpu.async_copy(rel_im.at[pl.ds(base, CH2)], rim, sem),
        ]
        for cp in cps:
            cp.wait()
        for g in range(CH2 // L):
            rows = lax.iota(jnp.int32, L) + (g * L)

            def dim_step(d4, acc, rows=rows):
                for u in range(4):
                    cols = d4 * 4 + jnp.full((L,), u, jnp.int32)
                    a = plsc.load_gather(hre, [rows, cols])
                    bb = plsc.load_gather(him, [rows, cols])
                    cr = plsc.load_gather(rre, [rows, cols])
                    ci = plsc.load_gather(rim, [rows, cols])
                    e = plsc.load_gather(tre, [rows, cols])
                    f = plsc.load_gather(tim, [rows, cols])
                    acc = acc + e * (a * cr - bb * ci) + f * (bb * cr + a * ci)
                return acc

            acc = lax.fori_loop(0, D // 4, dim_step,
                                jnp.zeros((L,), jnp.float32))
            scores[pl.ds(c * CH2 + g * L, L)] = acc
    pltpu.sync_copy(scores, out_hbm.at[pl.ds(wid * BPW, BPW)])


def _next_distinct(blocks, flags):
    n = blocks.shape[0]
    idxs = jnp.arange(n, dtype=jnp.int32)
    binf = jnp.where(flags == 1, idxs, jnp.int32(2**30))
    suffix_min = lax.cummin(binf, axis=0, reverse=True)
    nxt_idx = jnp.concatenate(
        [suffix_min[1:], jnp.full((1,), 2**30, jnp.int32)])
    return blocks[jnp.clip(nxt_idx, 0, n - 1)]


@functools.partial(jax.jit)
def kernel(h, r, t, entity_re, entity_im, relation_re, relation_im):
    h32 = h.astype(jnp.int32)
    t32 = t.astype(jnp.int32)
    r32 = r.astype(jnp.int32)
    keys = jnp.concatenate([h32, t32])
    order = jnp.argsort(keys).astype(jnp.int32)
    si = keys[order]
    bu = si >> 7
    flag = jnp.concatenate(
        [jnp.ones((1,), jnp.int32), (bu[1:] != bu[:-1]).astype(jnp.int32)])
    pf = order + flag * 65536
    nx = jnp.clip(_next_distinct(bu, flag), 0, NEB - 1)

    rorder = jnp.argsort(r32).astype(jnp.int32)
    sr = r32[rorder]
    rbu = sr >> 7
    rflag = jnp.concatenate(
        [jnp.ones((1,), jnp.int32), (rbu[1:] != rbu[:-1]).astype(jnp.int32)])
    rpf = rorder + rflag * 65536
    rnx = jnp.clip(_next_distinct(rbu, rflag), 0, 7)

    ere_t = entity_re.T
    eim_t = entity_im.T
    rre_t = relation_re.T
    rim_t = relation_im.T

    mesh = plsc.VectorSubcoreMesh(
        core_axis_name="c", subcore_axis_name="s", num_cores=NC,
        num_subcores=NS)
    p1 = pl.kernel(
        _phase1_body,
        out_type=(
            jax.ShapeDtypeStruct((2 * B, 2 * D), jnp.float32),
            jax.ShapeDtypeStruct((2 * B, 2 * D), jnp.float32),
            jax.ShapeDtypeStruct((B, 2 * D), jnp.float32),
            jax.ShapeDtypeStruct((B, 2 * D), jnp.float32),
        ),
        mesh=mesh,
        scratch_types=[
            pltpu.VMEM((EU,), jnp.int32),
            pltpu.VMEM((EU,), jnp.int32),
            pltpu.VMEM((EU,), jnp.int32),
            pltpu.VMEM((RU,), jnp.int32),
            pltpu.VMEM((RU,), jnp.int32),
            pltpu.VMEM((RU,), jnp.int32),
            pltpu.VMEM((2, D, EB), jnp.float32),
            pltpu.VMEM((2, D, EB), jnp.float32),
            pltpu.VMEM((64 * 2 * D,), jnp.float32),
            pltpu.VMEM((64 * 2 * D,), jnp.float32),
            pltpu.SemaphoreType.DMA,
            pltpu.SemaphoreType.DMA,
        ],
        compiler_params=pltpu.CompilerParams(
            needs_layout_passes=False, disable_bounds_checks=True),
    )
    rows_re, rows_im, rel_re, rel_im = p1(
        si, pf, nx, sr, rpf, rnx, ere_t, eim_t, rre_t, rim_t)

    p2 = pl.kernel(
        _phase2_body,
        out_type=jax.ShapeDtypeStruct((B,), jnp.float32),
        mesh=mesh,
        scratch_types=[
            pltpu.VMEM((64, 2 * D), jnp.float32),
            pltpu.VMEM((64, 2 * D), jnp.float32),
            pltpu.VMEM((64, 2 * D), jnp.float32),
            pltpu.VMEM((64, 2 * D), jnp.float32),
            pltpu.VMEM((64, 2 * D), jnp.float32),
            pltpu.VMEM((64, 2 * D), jnp.float32),
            pltpu.VMEM((BPW,), jnp.float32),
            pltpu.SemaphoreType.DMA,
        ],
        compiler_params=pltpu.CompilerParams(needs_layout_passes=False),
    )
    return p2(rows_re, rows_im, rel_re, rel_im)


# speedup vs baseline: 1.1341x; 1.1168x over previous
import functools

import jax
import jax.numpy as jnp
from jax import lax
from jax.experimental import pallas as pl
from jax.experimental.pallas import tpu as pltpu
from jax.experimental.pallas import tpu_sc as plsc

B = 16384
D = 64
NC = 2
NS = 16
L = 16
NW = NC * NS
BPW = B // NW
CH = 32
NCH = BPW // CH


def _sc_body(h_hbm, r_hbm, t_hbm, ere_hbm, eim_hbm, rre_hbm, rim_hbm,
             out_hbm, hidx, ridx, tidx, bufsA, bufsB, scores, semA, semB):
    wid = lax.axis_index("s") * NC + lax.axis_index("c")
    pltpu.sync_copy(h_hbm.at[wid], hidx)
    pltpu.sync_copy(r_hbm.at[wid], ridx)
    pltpu.sync_copy(t_hbm.at[wid], tidx)

    def issue(c, bufs, sem):
        hre, him, tre, tim, rre, rim = bufs
        for g in range(CH // L):
            base = c * CH + g * L
            hv = hidx[pl.ds(base, L)]
            tv = tidx[pl.ds(base, L)]
            rv = ridx[pl.ds(base, L)]
            hj, hs = hv >> 3, hv & 7
            tj, ts = tv >> 3, tv & 7
            rj, rs = rv >> 3, rv & 7
            for k in range(L):
                p = g * L + k
                pltpu.async_copy(ere_hbm.at[hj[k], hs[k]], hre.at[p], sem)
                pltpu.async_copy(eim_hbm.at[hj[k], hs[k]], him.at[p], sem)
                pltpu.async_copy(ere_hbm.at[tj[k], ts[k]], tre.at[p], sem)
                pltpu.async_copy(eim_hbm.at[tj[k], ts[k]], tim.at[p], sem)
                pltpu.async_copy(rre_hbm.at[rj[k], rs[k]], rre.at[p], sem)
                pltpu.async_copy(rim_hbm.at[rj[k], rs[k]], rim.at[p], sem)

    def drain(bufs, sem):
        for buf in bufs:
            pltpu.make_async_copy(
                ere_hbm.at[pl.ds(0, CH), 0], buf, sem).wait()

    def compute(c, bufs):
        hre, him, tre, tim, rre, rim = bufs
        for g in range(CH // L):
            rows = lax.iota(jnp.int32, L) + (g * L)

            def dim_step(d4, acc, rows=rows):
                for u in range(4):
                    cols = d4 * 4 + jnp.full((L,), u, jnp.int32)
                    a = plsc.load_gather(hre, [rows, cols])
                    bb = plsc.load_gather(him, [rows, cols])
                    cr = plsc.load_gather(rre, [rows, cols])
                    ci = plsc.load_gather(rim, [rows, cols])
                    e = plsc.load_gather(tre, [rows, cols])
                    f = plsc.load_gather(tim, [rows, cols])
                    acc = acc + e * (a * cr - bb * ci) + f * (bb * cr + a * ci)
                return acc

            acc = lax.fori_loop(0, D // 4, dim_step,
                                jnp.zeros((L,), jnp.float32))
            scores[pl.ds(c * CH + g * L, L)] = acc

    issue(0, bufsA, semA)

    def pair(m, carry):
        c0 = m * 2
        issue(c0 + 1, bufsB, semB)
        drain(bufsA, semA)
        compute(c0, bufsA)

        @pl.when(m < NCH // 2 - 1)
        def _():
            issue(c0 + 2, bufsA, semA)

        drain(bufsB, semB)
        compute(c0 + 1, bufsB)
        return carry

    lax.fori_loop(0, NCH // 2, pair, 0)
    pltpu.sync_copy(scores, out_hbm.at[pl.ds(wid * BPW, BPW)])


@functools.partial(jax.jit)
def kernel(h, r, t, entity_re, entity_im, relation_re, relation_im):
    h2 = h.astype(jnp.int32).reshape(NW, BPW)
    r2 = r.astype(jnp.int32).reshape(NW, BPW)
    t2 = t.astype(jnp.int32).reshape(NW, BPW)
    ere = entity_re.reshape(-1, 8, D)
    eim = entity_im.reshape(-1, 8, D)
    rre = relation_re.reshape(-1, 8, D)
    rim = relation_im.reshape(-1, 8, D)
    mesh = plsc.VectorSubcoreMesh(
        core_axis_name="c", subcore_axis_name="s", num_cores=NC,
        num_subcores=NS)
    widths = (D, D, D, D, D, D)
    run = pl.kernel(
        _sc_body,
        out_type=jax.ShapeDtypeStruct((B,), jnp.float32),
        mesh=mesh,
        scratch_types=[
            pltpu.VMEM((BPW,), jnp.int32),
            pltpu.VMEM((BPW,), jnp.int32),
            pltpu.VMEM((BPW,), jnp.int32),
            [pltpu.VMEM((CH, w), jnp.float32) for w in widths],
            [pltpu.VMEM((CH, w), jnp.float32) for w in widths],
            pltpu.VMEM((BPW,), jnp.float32),
            pltpu.SemaphoreType.DMA,
            pltpu.SemaphoreType.DMA,
        ],
        compiler_params=pltpu.CompilerParams(needs_layout_passes=False),
    )
    return run(h2, r2, t2, ere, eim, rre, rim)
